# baseline (speedup 1.0000x reference)
import jax
import jax.numpy as jnp
from jax import lax
from jax.experimental import pallas as pl
from jax.experimental.pallas import tpu as pltpu
from jax.experimental.pallas import tpu_sc as plsc

N = 10000
D = 256
E = 160000

NPAD = 10240
EPAD = 163840
CHUNK = EPAD // 32
NBATCH = CHUNK // 16
HALF = N // 2
HALFPAD = 5120
DUMMY_ROW = HALFPAD - 1

_MESH = plsc.VectorSubcoreMesh(
    core_axis_name="c", subcore_axis_name="s", num_cores=2, num_subcores=16)


def _tc_body(f_ref, w_ref, b_ref, wa1_ref, ba1_ref, wa2_ref, ba2_ref,
             h_ref, a12_ref):
    f = f_ref[...]
    w = w_ref[...]
    h = lax.dot_general(f, w, (((1,), (1,)), ((), ())),
                        precision=lax.Precision.HIGHEST,
                        preferred_element_type=jnp.float32)
    h = h + b_ref[...]
    h_ref[...] = h
    a1 = lax.dot_general(wa1_ref[...], h, (((1,), (1,)), ((), ())),
                         precision=lax.Precision.HIGHEST,
                         preferred_element_type=jnp.float32)
    a2 = lax.dot_general(wa2_ref[...], h, (((1,), (1,)), ((), ())),
                         precision=lax.Precision.HIGHEST,
                         preferred_element_type=jnp.float32)
    a12_ref[...] = jnp.concatenate(
        [a1 + ba1_ref[...], a2 + ba2_ref[...]], axis=0)


def _tc_linear(feat_pad, W_layer, b2, wa1, ba1, wa2, ba2):
    blk = 512
    grid = NPAD // blk
    return pl.pallas_call(
        _tc_body,
        grid=(grid,),
        in_specs=[
            pl.BlockSpec((blk, D), lambda i: (i, 0)),
            pl.BlockSpec((D, D), lambda i: (0, 0)),
            pl.BlockSpec((1, D), lambda i: (0, 0)),
            pl.BlockSpec((1, D), lambda i: (0, 0)),
            pl.BlockSpec((1, 1), lambda i: (0, 0)),
            pl.BlockSpec((1, D), lambda i: (0, 0)),
            pl.BlockSpec((1, 1), lambda i: (0, 0)),
        ],
        out_specs=[
            pl.BlockSpec((blk, D), lambda i: (i, 0)),
            pl.BlockSpec((2, blk), lambda i: (0, i)),
        ],
        out_shape=[
            jax.ShapeDtypeStruct((NPAD, D), jnp.float32),
            jax.ShapeDtypeStruct((2, NPAD), jnp.float32),
        ],
    )(feat_pad, W_layer, b2, wa1, ba1, wa2, ba2)


def _zero_1d(ref, nvec):
    z = jnp.zeros((16,), jnp.float32)

    def body(i, _):
        ref[pl.ds(i * 16, 16)] = z
        return 0

    lax.fori_loop(0, nvec, body, 0)


def _sc_phase1_body(srcp, dstp, a12, vexp_hbm, vsump_hbm,
                    src_v, dst_v, a1_v, a2_v, vexp_v, vsum_v, tmp_v, acc_v,
                    bigbuf):
    c = lax.axis_index("c")
    s = lax.axis_index("s")
    wid = s * 2 + c
    base = wid * CHUNK

    pltpu.sync_copy(srcp.at[pl.ds(base, CHUNK)], src_v)
    pltpu.sync_copy(dstp.at[pl.ds(base, CHUNK)], dst_v)
    pltpu.sync_copy(a12.at[pl.ds(0, NPAD)], a1_v)
    pltpu.sync_copy(a12.at[pl.ds(NPAD, NPAD)], a2_v)
    _zero_1d(vsum_v, NPAD // 16)

    def body(i, _):
        s16 = src_v[pl.ds(i * 16, 16)]
        d16 = dst_v[pl.ds(i * 16, 16)]
        a1g = plsc.load_gather(a1_v, [s16])
        a2g = plsc.load_gather(a2_v, [d16])
        v = a1g + a2g
        v = jnp.maximum(v, v * jnp.float32(0.01))
        e = jnp.exp(v)
        vexp_v[pl.ds(i * 16, 16)] = e
        plsc.addupdate_scatter(vsum_v, [s16], e)
        return 0

    lax.fori_loop(0, NBATCH, body, 0)

    pltpu.sync_copy(vexp_v, vexp_hbm.at[pl.ds(base, CHUNK)])

    pltpu.sync_copy(vsum_v, bigbuf.at[s])
    plsc.subcore_barrier()

    seg = NPAD // 16
    _zero_1d(acc_v, seg // 16)
    for r in range(16):
        pltpu.sync_copy(bigbuf.at[r, pl.ds(s * seg, seg)], tmp_v)

        def addb(j, _):
            acc_v[pl.ds(j * 16, 16)] = (acc_v[pl.ds(j * 16, 16)]
                                        + tmp_v[pl.ds(j * 16, 16)])
            return 0

        lax.fori_loop(0, seg // 16, addb, 0)
    pltpu.sync_copy(acc_v, vsump_hbm.at[pl.ds(c * NPAD + s * seg, seg)])


def _sc_phase1(srcp, dstp, a12):
    return pl.kernel(
        _sc_phase1_body,
        out_type=[
            jax.ShapeDtypeStruct((EPAD,), jnp.float32),
            jax.ShapeDtypeStruct((2 * NPAD,), jnp.float32),
        ],
        mesh=_MESH,
        compiler_params=pltpu.CompilerParams(needs_layout_passes=False),
        scratch_types=[
            pltpu.VMEM((CHUNK,), jnp.int32),
            pltpu.VMEM((CHUNK,), jnp.int32),
            pltpu.VMEM((NPAD,), jnp.float32),
            pltpu.VMEM((NPAD,), jnp.float32),
            pltpu.VMEM((CHUNK,), jnp.float32),
            pltpu.VMEM((NPAD,), jnp.float32),
            pltpu.VMEM((NPAD // 16,), jnp.float32),
            pltpu.VMEM((NPAD // 16,), jnp.float32),
            pltpu.VMEM_SHARED((16, NPAD), jnp.float32),
        ],
    )(srcp, dstp, a12)


ROWS_PER_TILE = NPAD // 32
SCHUNK = 4096
SNB = SCHUNK // 16
GB = 32
PCAP = SCHUNK + 2 * GB


def _sc_phase2_body(srcp, dstp, vexp, vsump, h, outp,
                    src_v, dst_v, vexp_v, vsum_loc, vsum2_loc,
                    psrc, pdst, pval, gbuf, out_loc,
                    sem0, sem1, semc):
    c = lax.axis_index("c")
    s = lax.axis_index("s")
    wid = s * 2 + c
    rlo = wid * ROWS_PER_TILE

    pltpu.sync_copy(vsump.at[pl.ds(rlo, ROWS_PER_TILE)],
                    vsum_loc.at[pl.ds(0, ROWS_PER_TILE)])
    pltpu.sync_copy(vsump.at[pl.ds(NPAD + rlo, ROWS_PER_TILE)], vsum2_loc)

    def addb(j, _):
        vsum_loc[pl.ds(j * 16, 16)] = (vsum_loc[pl.ds(j * 16, 16)]
                                       + vsum2_loc[pl.ds(j * 16, 16)])
        return 0

    lax.fori_loop(0, ROWS_PER_TILE // 16, addb, 0)
    vsum_loc[pl.ds(ROWS_PER_TILE, 16)] = jnp.ones((16,), jnp.float32)

    def zrow(i, _):
        out_loc[i // 16, pl.ds((i % 16) * 16, 16)] = (
            jnp.zeros((16,), jnp.float32))
        return 0

    lax.fori_loop(0, (ROWS_PER_TILE + 1) * 16, zrow, 0)

    zero16 = jnp.zeros((16,), jnp.float32)
    dummy16 = jnp.full((16,), ROWS_PER_TILE, jnp.int32)
    zero16i = jnp.zeros((16,), jnp.int32)

    def issue(g, buf, sem):
        for q in range(GB // 16):
            d16 = pdst[pl.ds(g * GB + q * 16, 16)]
            pltpu.async_copy(h.at[d16], buf.at[pl.ds(q * 16, 16)], sem)

    def wait(g, buf, sem):
        for q in range(GB // 16):
            d16 = pdst[pl.ds(g * GB + q * 16, 16)]
            pltpu.make_async_copy(
                h.at[d16], buf.at[pl.ds(q * 16, 16)], sem).wait()

    def process(g, buf):
        for q in range(GB // 16):
            r16 = psrc[pl.ds(g * GB + q * 16, 16)]
            e16 = pval[pl.ds(g * GB + q * 16, 16)]
            vs = plsc.load_gather(vsum_loc, [r16])
            attn = e16 / vs
            for r in range(16):
                a = attn[r]
                row = r16[r]
                vals = [buf[q * 16 + r, pl.ds(j * 16, 16)]
                        for j in range(16)]
                scaled = [v * a for v in vals]
                for j in range(16):
                    plsc.addupdate(out_loc.at[row, pl.ds(j * 16, 16)],
                                   scaled[j])

    def chunk_body(k, _):
        base = k * SCHUNK
        c1 = pltpu.async_copy(srcp.at[pl.ds(base, SCHUNK)], src_v, semc)
        c2 = pltpu.async_copy(dstp.at[pl.ds(base, SCHUNK)], dst_v, semc)
        c3 = pltpu.async_copy(vexp.at[pl.ds(base, SCHUNK)], vexp_v, semc)
        c1.wait()
        c2.wait()
        c3.wait()

        def scan(i, cnt):
            s16 = src_v[pl.ds(i * 16, 16)]
            sl = s16 - rlo
            m = (sl >= 0) & (sl < ROWS_PER_TILE) & (s16 < N)
            npop = plsc.all_reduce_population_count(m)[0]

            @pl.when(npop > 0)
            def _():
                slc = jnp.where(m, sl, 0)
                d16 = dst_v[pl.ds(i * 16, 16)]
                e16 = vexp_v[pl.ds(i * 16, 16)]
                plsc.store_compressed(psrc.at[pl.ds(cnt, 16)], slc, mask=m)
                plsc.store_compressed(pdst.at[pl.ds(cnt, 16)], d16, mask=m)
                plsc.store_compressed(pval.at[pl.ds(cnt, 16)], e16, mask=m)

            return cnt + npop

        cnt = lax.fori_loop(0, SNB, scan, jnp.int32(0), unroll=2)

        for t in range(2 * GB // 16):
            psrc[pl.ds(cnt + t * 16, 16)] = dummy16
            pdst[pl.ds(cnt + t * 16, 16)] = zero16i
            pval[pl.ds(cnt + t * 16, 16)] = zero16

        nb2 = (cnt + 2 * GB - 1) // (2 * GB)

        issue(0, gbuf.at[0], sem0)

        def pair(t, _):
            g0 = 2 * t
            g1 = g0 + 1
            issue(g1, gbuf.at[1], sem1)
            wait(g0, gbuf.at[0], sem0)
            process(g0, gbuf.at[0])

            @pl.when(t + 1 < nb2)
            def _():
                issue(g0 + 2, gbuf.at[0], sem0)

            wait(g1, gbuf.at[1], sem1)
            process(g1, gbuf.at[1])
            return 0

        lax.fori_loop(0, nb2, pair, 0)
        return 0

    lax.fori_loop(0, EPAD // SCHUNK, chunk_body, 0)


    pltpu.sync_copy(out_loc.at[pl.ds(0, ROWS_PER_TILE)],
                    outp.at[pl.ds(rlo, ROWS_PER_TILE)])


def _sc_phase2(srcp, dstp, vexp, vsump, h):
    return pl.kernel(
        _sc_phase2_body,
        out_type=jax.ShapeDtypeStruct((NPAD, D), jnp.float32),
        mesh=_MESH,
        compiler_params=pltpu.CompilerParams(needs_layout_passes=False),
        scratch_types=[
            pltpu.VMEM((SCHUNK,), jnp.int32),
            pltpu.VMEM((SCHUNK,), jnp.int32),
            pltpu.VMEM((SCHUNK,), jnp.float32),
            pltpu.VMEM((ROWS_PER_TILE + 16,), jnp.float32),
            pltpu.VMEM((ROWS_PER_TILE,), jnp.float32),
            pltpu.VMEM((PCAP,), jnp.int32),
            pltpu.VMEM((PCAP,), jnp.int32),
            pltpu.VMEM((PCAP,), jnp.float32),
            pltpu.VMEM((2, GB, D), jnp.float32),
            pltpu.VMEM((ROWS_PER_TILE + 1, D), jnp.float32),
            pltpu.SemaphoreType.DMA,
            pltpu.SemaphoreType.DMA,
            pltpu.SemaphoreType.DMA,
        ],
    )(srcp, dstp, vexp, vsump, h)


def kernel(features, adj_indices, adj_values, W_layer, b_layer,
           W_a1, b_a1, W_a2, b_a2, w_a3, b_a3):
    del adj_values, w_a3, b_a3
    feat_pad = jnp.zeros((NPAD, D), jnp.float32).at[:N].set(features)
    src = adj_indices[0, :].astype(jnp.int32)
    dst = adj_indices[1, :].astype(jnp.int32)
    srcp = jnp.full((EPAD,), N, jnp.int32).at[:E].set(src)
    dstp = jnp.zeros((EPAD,), jnp.int32).at[:E].set(dst)

    h, a12 = _tc_linear(feat_pad, W_layer, b_layer.reshape(1, D),
                        W_a1.reshape(1, D), b_a1.reshape(1, 1),
                        W_a2.reshape(1, D), b_a2.reshape(1, 1))
    vexp, vsump = _sc_phase1(srcp, dstp, a12.reshape(2 * NPAD))
    out_pad = _sc_phase2(srcp, dstp, vexp, vsump, h)
    return out_pad[:N]


# speedup vs baseline: 2.1785x; 2.1785x over previous
import jax
import jax.numpy as jnp
from jax import lax
from jax.experimental import pallas as pl
from jax.experimental.pallas import tpu as pltpu
from jax.experimental.pallas import tpu_sc as plsc

N = 10000
D = 256
E = 160000

NPAD = 10240
EPAD = 163840
CHUNK = EPAD // 32
NBATCH = CHUNK // 16
HALF = N // 2
HALFPAD = 5120
DUMMY_ROW = HALFPAD - 1

_MESH = plsc.VectorSubcoreMesh(
    core_axis_name="c", subcore_axis_name="s", num_cores=2, num_subcores=16)


def _tc_body(f_ref, w_ref, b_ref, wa1_ref, ba1_ref, wa2_ref, ba2_ref,
             h_ref, a12_ref):
    f = f_ref[...]
    w = w_ref[...]
    h = lax.dot_general(f, w, (((1,), (1,)), ((), ())),
                        precision=lax.Precision.HIGHEST,
                        preferred_element_type=jnp.float32)
    h = h + b_ref[...]
    h_ref[...] = h
    a1 = lax.dot_general(wa1_ref[...], h, (((1,), (1,)), ((), ())),
                         precision=lax.Precision.HIGHEST,
                         preferred_element_type=jnp.float32)
    a2 = lax.dot_general(wa2_ref[...], h, (((1,), (1,)), ((), ())),
                         precision=lax.Precision.HIGHEST,
                         preferred_element_type=jnp.float32)
    a12_ref[...] = jnp.concatenate(
        [a1 + ba1_ref[...], a2 + ba2_ref[...]], axis=0)


def _tc_linear(feat_pad, W_layer, b2, wa1, ba1, wa2, ba2):
    blk = 512
    grid = NPAD // blk
    return pl.pallas_call(
        _tc_body,
        grid=(grid,),
        in_specs=[
            pl.BlockSpec((blk, D), lambda i: (i, 0)),
            pl.BlockSpec((D, D), lambda i: (0, 0)),
            pl.BlockSpec((1, D), lambda i: (0, 0)),
            pl.BlockSpec((1, D), lambda i: (0, 0)),
            pl.BlockSpec((1, 1), lambda i: (0, 0)),
            pl.BlockSpec((1, D), lambda i: (0, 0)),
            pl.BlockSpec((1, 1), lambda i: (0, 0)),
        ],
        out_specs=[
            pl.BlockSpec((blk, D), lambda i: (i, 0)),
            pl.BlockSpec((2, blk), lambda i: (0, i)),
        ],
        out_shape=[
            jax.ShapeDtypeStruct((NPAD, D), jnp.float32),
            jax.ShapeDtypeStruct((2, NPAD), jnp.float32),
        ],
    )(feat_pad, W_layer, b2, wa1, ba1, wa2, ba2)


def _zero_1d(ref, nvec):
    z = jnp.zeros((16,), jnp.float32)

    def body(i, _):
        ref[pl.ds(i * 16, 16)] = z
        return 0

    lax.fori_loop(0, nvec, body, 0)


def _sc_phase1_body(srcp, dstp, a12, vexp_hbm, vsump_hbm,
                    src_v, dst_v, a1_v, a2_v, vexp_v, vsum_v, tmp_v, acc_v,
                    bigbuf):
    c = lax.axis_index("c")
    s = lax.axis_index("s")
    wid = s * 2 + c
    base = wid * CHUNK

    pltpu.sync_copy(srcp.at[pl.ds(base, CHUNK)], src_v)
    pltpu.sync_copy(dstp.at[pl.ds(base, CHUNK)], dst_v)
    pltpu.sync_copy(a12.at[pl.ds(0, NPAD)], a1_v)
    pltpu.sync_copy(a12.at[pl.ds(NPAD, NPAD)], a2_v)
    _zero_1d(vsum_v, NPAD // 16)

    def body(i, _):
        s16 = src_v[pl.ds(i * 16, 16)]
        d16 = dst_v[pl.ds(i * 16, 16)]
        a1g = plsc.load_gather(a1_v, [s16])
        a2g = plsc.load_gather(a2_v, [d16])
        v = a1g + a2g
        v = jnp.maximum(v, v * jnp.float32(0.01))
        e = jnp.exp(v)
        vexp_v[pl.ds(i * 16, 16)] = e
        plsc.addupdate_scatter(vsum_v, [s16], e)
        return 0

    lax.fori_loop(0, NBATCH, body, 0)

    pltpu.sync_copy(vexp_v, vexp_hbm.at[pl.ds(base, CHUNK)])

    pltpu.sync_copy(vsum_v, bigbuf.at[s])
    plsc.subcore_barrier()

    seg = NPAD // 16
    _zero_1d(acc_v, seg // 16)
    for r in range(16):
        pltpu.sync_copy(bigbuf.at[r, pl.ds(s * seg, seg)], tmp_v)

        def addb(j, _):
            acc_v[pl.ds(j * 16, 16)] = (acc_v[pl.ds(j * 16, 16)]
                                        + tmp_v[pl.ds(j * 16, 16)])
            return 0

        lax.fori_loop(0, seg // 16, addb, 0)
    pltpu.sync_copy(acc_v, vsump_hbm.at[pl.ds(c * NPAD + s * seg, seg)])


def _sc_phase1(srcp, dstp, a12):
    return pl.kernel(
        _sc_phase1_body,
        out_type=[
            jax.ShapeDtypeStruct((EPAD,), jnp.float32),
            jax.ShapeDtypeStruct((2 * NPAD,), jnp.float32),
        ],
        mesh=_MESH,
        compiler_params=pltpu.CompilerParams(needs_layout_passes=False),
        scratch_types=[
            pltpu.VMEM((CHUNK,), jnp.int32),
            pltpu.VMEM((CHUNK,), jnp.int32),
            pltpu.VMEM((NPAD,), jnp.float32),
            pltpu.VMEM((NPAD,), jnp.float32),
            pltpu.VMEM((CHUNK,), jnp.float32),
            pltpu.VMEM((NPAD,), jnp.float32),
            pltpu.VMEM((NPAD // 16,), jnp.float32),
            pltpu.VMEM((NPAD // 16,), jnp.float32),
            pltpu.VMEM_SHARED((16, NPAD), jnp.float32),
        ],
    )(srcp, dstp, a12)


ROWS_PER_TILE = NPAD // 32
SCHUNK = 5120
SNB = SCHUNK // 16
GB = 16
PCAP = SCHUNK + 2 * GB


def _sc_phase2_body(srcp, dstp, vexp, vsump, h, outp,
                    src_v, dst_v, vexp_v, vsum_loc, vsum2_loc,
                    psrc, pdst, pval, gbuf, out_loc,
                    sem0, sem1, semc):
    c = lax.axis_index("c")
    s = lax.axis_index("s")
    wid = s * 2 + c
    rlo = wid * ROWS_PER_TILE

    pltpu.sync_copy(vsump.at[pl.ds(rlo, ROWS_PER_TILE)],
                    vsum_loc.at[pl.ds(0, ROWS_PER_TILE)])
    pltpu.sync_copy(vsump.at[pl.ds(NPAD + rlo, ROWS_PER_TILE)], vsum2_loc)

    def addb(j, _):
        vsum_loc[pl.ds(j * 16, 16)] = (vsum_loc[pl.ds(j * 16, 16)]
                                       + vsum2_loc[pl.ds(j * 16, 16)])
        return 0

    lax.fori_loop(0, ROWS_PER_TILE // 16, addb, 0)
    vsum_loc[pl.ds(ROWS_PER_TILE, 16)] = jnp.ones((16,), jnp.float32)

    def zrow(i, _):
        out_loc[i // 16, pl.ds((i % 16) * 16, 16)] = (
            jnp.zeros((16,), jnp.float32))
        return 0

    lax.fori_loop(0, (ROWS_PER_TILE + 1) * 16, zrow, 0)

    zero16 = jnp.zeros((16,), jnp.float32)
    dummy16 = jnp.full((16,), ROWS_PER_TILE, jnp.int32)
    zero16i = jnp.zeros((16,), jnp.int32)

    def issue(g, buf, sem):
        for q in range(GB // 16):
            d16 = pdst[pl.ds(g * GB + q * 16, 16)]
            pltpu.async_copy(h.at[d16], buf.at[pl.ds(q * 16, 16)], sem)

    def wait(g, buf, sem):
        for q in range(GB // 16):
            d16 = pdst[pl.ds(g * GB + q * 16, 16)]
            pltpu.make_async_copy(
                h.at[d16], buf.at[pl.ds(q * 16, 16)], sem).wait()

    def process(g, buf):
        for q in range(GB // 16):
            r16 = psrc[pl.ds(g * GB + q * 16, 16)]
            e16 = pval[pl.ds(g * GB + q * 16, 16)]
            vs = plsc.load_gather(vsum_loc, [r16])
            attn = e16 / vs
            for r in range(16):
                a = attn[r]
                row = r16[r]
                vals = [buf[q * 16 + r, pl.ds(j * 16, 16)]
                        for j in range(16)]
                scaled = [v * a for v in vals]
                for j in range(16):
                    plsc.addupdate(out_loc.at[row, pl.ds(j * 16, 16)],
                                   scaled[j])

    def chunk_body(k, _):
        base = k * SCHUNK
        c1 = pltpu.async_copy(srcp.at[pl.ds(base, SCHUNK)], src_v, semc)
        c2 = pltpu.async_copy(dstp.at[pl.ds(base, SCHUNK)], dst_v, semc)
        c3 = pltpu.async_copy(vexp.at[pl.ds(base, SCHUNK)], vexp_v, semc)
        c1.wait()
        c2.wait()
        c3.wait()

        def scan(i, cnt):
            s16 = src_v[pl.ds(i * 16, 16)]
            sl = s16 - rlo
            m = (sl >= 0) & (sl < ROWS_PER_TILE) & (s16 < N)
            d16 = dst_v[pl.ds(i * 16, 16)]
            e16 = vexp_v[pl.ds(i * 16, 16)]
            slc = jnp.where(m, sl, 0)
            npop = plsc.all_reduce_population_count(m)[0]
            plsc.store_compressed(psrc.at[pl.ds(cnt, 16)], slc, mask=m)
            plsc.store_compressed(pdst.at[pl.ds(cnt, 16)], d16, mask=m)
            plsc.store_compressed(pval.at[pl.ds(cnt, 16)], e16, mask=m)
            return cnt + npop

        cnt = lax.fori_loop(0, SNB, scan, jnp.int32(0), unroll=4)

        for t in range(2 * GB // 16):
            psrc[pl.ds(cnt + t * 16, 16)] = dummy16
            pdst[pl.ds(cnt + t * 16, 16)] = zero16i
            pval[pl.ds(cnt + t * 16, 16)] = zero16

        nb2 = (cnt + 2 * GB - 1) // (2 * GB)

        issue(0, gbuf.at[0], sem0)

        def pair(t, _):
            g0 = 2 * t
            g1 = g0 + 1
            issue(g1, gbuf.at[1], sem1)
            wait(g0, gbuf.at[0], sem0)
            process(g0, gbuf.at[0])

            @pl.when(t + 1 < nb2)
            def _():
                issue(g0 + 2, gbuf.at[0], sem0)

            wait(g1, gbuf.at[1], sem1)
            process(g1, gbuf.at[1])
            return 0

        lax.fori_loop(0, nb2, pair, 0)
        return 0

    lax.fori_loop(0, EPAD // SCHUNK, chunk_body, 0)


    pltpu.sync_copy(out_loc.at[pl.ds(0, ROWS_PER_TILE)],
                    outp.at[pl.ds(rlo, ROWS_PER_TILE)])


def _sc_phase2(srcp, dstp, vexp, vsump, h):
    return pl.kernel(
        _sc_phase2_body,
        out_type=jax.ShapeDtypeStruct((NPAD, D), jnp.float32),
        mesh=_MESH,
        compiler_params=pltpu.CompilerParams(needs_layout_passes=False),
        scratch_types=[
            pltpu.VMEM((SCHUNK,), jnp.int32),
            pltpu.VMEM((SCHUNK,), jnp.int32),
            pltpu.VMEM((SCHUNK,), jnp.float32),
            pltpu.VMEM((ROWS_PER_TILE + 16,), jnp.float32),
            pltpu.VMEM((ROWS_PER_TILE,), jnp.float32),
            pltpu.VMEM((PCAP,), jnp.int32),
            pltpu.VMEM((PCAP,), jnp.int32),
            pltpu.VMEM((PCAP,), jnp.float32),
            pltpu.VMEM((2, GB, D), jnp.float32),
            pltpu.VMEM((ROWS_PER_TILE + 1, D), jnp.float32),
            pltpu.SemaphoreType.DMA,
            pltpu.SemaphoreType.DMA,
            pltpu.SemaphoreType.DMA,
        ],
    )(srcp, dstp, vexp, vsump, h)


def kernel(features, adj_indices, adj_values, W_layer, b_layer,
           W_a1, b_a1, W_a2, b_a2, w_a3, b_a3):
    del adj_values, w_a3, b_a3
    feat_pad = jnp.zeros((NPAD, D), jnp.float32).at[:N].set(features)
    src = adj_indices[0, :].astype(jnp.int32)
    dst = adj_indices[1, :].astype(jnp.int32)
    srcp = jnp.full((EPAD,), N, jnp.int32).at[:E].set(src)
    dstp = jnp.zeros((EPAD,), jnp.int32).at[:E].set(dst)

    h, a12 = _tc_linear(feat_pad, W_layer, b_layer.reshape(1, D),
                        W_a1.reshape(1, D), b_a1.reshape(1, 1),
                        W_a2.reshape(1, D), b_a2.reshape(1, 1))
    vexp, vsump = _sc_phase1(srcp, dstp, a12.reshape(2 * NPAD))
    out_pad = _sc_phase2(srcp, dstp, vexp, vsump, h)
    return out_pad[:N]
